# baseline (speedup 1.0000x reference)
import jax
import jax.numpy as jnp
from jax import lax
from jax.experimental import pallas as pl
from jax.experimental.pallas import tpu as pltpu
from jax.experimental.pallas import tpu_sc as plsc

N = 100000
E = 100000
D = 128
N_TOPIC = 10000

SEG = 100096
E4P = 4 * SEG
TS = E4P // 32
NV = TS // 16
CH = 8192
NCH = 50
POOL = NCH * CH
NK = 13
BKT = 32 * 13376
BKTX = BKT + 16
PADPOS = E4P
DSTX = E4P + 64
TEXT = 400160
PAD_DST = 2_000_000
EP = 102400
EPT = EP // 32
GB = 64

_f32 = jnp.float32
_i32 = jnp.int32


def _transform_kernel(hl_ref, hb_ref, hc_ref, w_ref, out_ref):
    w = w_ref[...]
    dn = (((1,), (1,)), ((), ()))
    out_ref[0] = lax.dot_general(hl_ref[...], w[:, 0:128], dn)
    out_ref[1] = lax.dot_general(hb_ref[...], w[:, 128:256], dn)
    out_ref[2] = lax.dot_general(hc_ref[...], w[:, 256:384], dn)
    out_ref[3] = lax.dot_general(hc_ref[...], w[:, 384:512], dn)


def _topic_kernel(ht_ref, w_ref, b_ref, out_ref):
    dn = (((1,), (1,)), ((), ()))
    out_ref[...] = lax.dot_general(ht_ref[...], w_ref[...][:, 512:640], dn) + b_ref[...]


def _div_kernel(s_ref, c_ref, out_ref):
    out_ref[...] = s_ref[...] / jnp.maximum(c_ref[...], 1.0)[:, None]


def _transforms(h_lt, h_bv, h_c, h_topic, W, b):
    RB = 1000
    t4 = pl.pallas_call(
        _transform_kernel,
        grid=(N // RB,),
        in_specs=[
            pl.BlockSpec((RB, D), lambda i: (i, 0)),
            pl.BlockSpec((RB, D), lambda i: (i, 0)),
            pl.BlockSpec((RB, D), lambda i: (i, 0)),
            pl.BlockSpec((D, 5 * D), lambda i: (0, 0)),
        ],
        out_specs=pl.BlockSpec((4, RB, D), lambda i: (0, i, 0)),
        out_shape=jax.ShapeDtypeStruct((4, N, D), _f32),
    )(h_lt, h_bv, h_c, W)
    t5 = pl.pallas_call(
        _topic_kernel,
        grid=(N_TOPIC // RB,),
        in_specs=[
            pl.BlockSpec((RB, D), lambda i: (i, 0)),
            pl.BlockSpec((D, 5 * D), lambda i: (0, 0)),
            pl.BlockSpec((D,), lambda i: (0,)),
        ],
        out_specs=pl.BlockSpec((RB, D), lambda i: (i, 0)),
        out_shape=jax.ShapeDtypeStruct((N_TOPIC, D), _f32),
    )(h_topic, W, b)
    return t4, t5


def _divide(pool_sums, cnts):
    RB = 1024
    return pl.pallas_call(
        _div_kernel,
        grid=(POOL // RB,),
        in_specs=[
            pl.BlockSpec((RB, D), lambda i: (i, 0)),
            pl.BlockSpec((RB,), lambda i: (i,)),
        ],
        out_specs=pl.BlockSpec((RB, D), lambda i: (i, 0)),
        out_shape=jax.ShapeDtypeStruct((POOL, D), _f32),
    )(pool_sums, cnts)


def _hist_body(dst_ref, hist_ref, dstbuf, hrow):
    c2 = lax.axis_index("c")
    s = lax.axis_index("s")
    t = c2 * 16 + s
    p = t // 8
    poff = p * N
    cb = 12 * p
    ones16 = jnp.full((16,), 1, _i32)
    zeros16 = jnp.full((16,), 0, _i32)

    pltpu.sync_copy(dst_ref.at[pl.ds(t * TS, TS)], dstbuf)

    cbkv = [jnp.full((16,), cb + k, _i32) for k in range(NK)]

    def sb(i, cnts):
        d = dstbuf[pl.ds(i * 16, 16)]
        cid = lax.shift_right_logical(d + poff, 13)
        out = []
        for k in range(NK):
            eq = cid == cbkv[k]
            out.append(cnts[k] + jnp.where(eq, ones16, zeros16))
        return tuple(out)
    cnts = lax.fori_loop(0, NV, sb, tuple(zeros16 for _ in range(NK)))
    for k in range(NK):
        hrow[pl.ds(k * 16, 16)] = cnts[k]
    pltpu.sync_copy(hrow, hist_ref.at[pl.ds(t * 16 * NK, 16 * NK)])


def _offsets(histbuf, own_store):
    def cloop(cidx, run):
        def tloop(tt, run2):
            pp = tt // 8
            kk = cidx - 12 * pp
            valid = (kk >= 0) & (kk < NK)
            kk_c = jnp.clip(kk, 0, NK - 1)
            hv = histbuf[pl.ds((tt * NK + kk_c) * 16, 16)]
            sm = hv[0]
            for L in range(1, 16):
                sm = sm + hv[L]
            cap = jnp.where(valid, ((sm + 63) >> 6) << 6, 0)
            own_store(cidx, tt, valid, run2, cap)
            return run2 + cap
        return lax.fori_loop(0, 32, tloop, run)
    lax.fori_loop(0, NCH, cloop, jnp.array(0, _i32))


def _place_body(dst_ref, hist_ref, bkt_ref,
                dstbuf, histbuf, stgp, stgd, bsem):
    c2 = lax.axis_index("c")
    s = lax.axis_index("s")
    t = c2 * 16 + s
    p = t // 8
    poff = p * N
    cb = 12 * p
    iota16 = lax.iota(_i32, 16)
    ones16 = jnp.full((16,), 1, _i32)
    zeros16 = jnp.full((16,), 0, _i32)

    pltpu.sync_copy(dst_ref.at[pl.ds(t * TS, TS)], dstbuf)
    pltpu.sync_copy(hist_ref, histbuf)

    def run_scoped_body(mb):
        def own_store(cidx, tt, valid, run2, cap):
            @pl.when(valid & (tt == t))
            def _():
                mb[cidx - cb] = run2
        _offsets(histbuf, own_store)

        cbkv = [jnp.full((16,), cb + k, _i32) for k in range(NK)]
        trashp = jnp.full((16,), BKT, _i32) + iota16
        base_pos = t * TS

        basev = []
        sums = []
        for k in range(NK):
            hv = histbuf[pl.ds((t * NK + k) * 16, 16)]
            bvec = jnp.full((16,), mb[k], _i32)
            run = mb[k]
            for L in range(1, 16):
                run = run + hv[L - 1]
                bvec = jnp.where(iota16 == jnp.full((16,), L, _i32),
                                 jnp.full((16,), 0, _i32) + run, bvec)
            sums.append(run + hv[15] - mb[k])
            basev.append(bvec)

        def do_vreg(i, u, ranks):
            d = dstbuf[pl.ds(i * 16, 16)]
            cid = lax.shift_right_logical(d + poff, 13)
            pos = trashp
            out = []
            for k in range(NK):
                eq = cid == cbkv[k]
                pos = jnp.where(eq, basev[k] + ranks[k], pos)
                out.append(ranks[k] + jnp.where(eq, ones16, zeros16))
            posn = base_pos + i * 16 + iota16
            sl = pl.ds(u * 16, 16)
            stgp[0, sl] = pos
            stgd[sl] = posn
            return tuple(out)

        def group(g, ranks):
            r = ranks
            for u in range(8):
                r = do_vreg(g * 8 + u, u, r)
            pltpu.async_copy(stgd, bkt_ref.at[stgp.at[0]], bsem).wait()
            return r
        ranks = lax.fori_loop(0, NV // 8, group,
                              tuple(zeros16 for _ in range(NK)))
        for u in range(6):
            ranks = do_vreg((NV // 8) * 8 + u, u, ranks)
        stgp[0, pl.ds(96, 16)] = trashp
        stgp[0, pl.ds(112, 16)] = trashp
        pltpu.async_copy(stgd, bkt_ref.at[stgp.at[0]], bsem).wait()

        for k in range(NK):
            padstart = mb[k] + sums[k]
            capend = mb[k] + (((sums[k] + 63) >> 6) << 6)
            for u in range(5):
                pv = jnp.full((16,), padstart, _i32) + u * 16 + iota16
                inr = pv < jnp.full((16,), capend, _i32)
                stgp[0, pl.ds(u * 16, 16)] = jnp.where(inr, pv, trashp)
                stgd[pl.ds(u * 16, 16)] = jnp.full((16,), PADPOS, _i32)
            for u in range(5, 8):
                stgp[0, pl.ds(u * 16, 16)] = trashp
            pltpu.async_copy(stgd, bkt_ref.at[stgp.at[0]], bsem).wait()

    pl.run_scoped(run_scoped_body, pltpu.SMEM((16,), _i32))


def _accum_body(dst_ref, hist_ref, bkt_ref, text_ref, pool_ref, cnt_ref,
                sh_s, sh_c,
                histbuf, pbuf, dbuf, rowibuf, dstlbuf, onesbuf,
                rdata, zrow, zcnt,
                gsem, rsem, ssem, osem, zsem):
    c2 = lax.axis_index("c")
    s = lax.axis_index("s")
    iota16 = lax.iota(_i32, 16)
    onesf = jnp.full((16,), 1.0, _f32)
    zerosf = jnp.full((16,), 0.0, _f32)
    th1 = jnp.full((16,), SEG, _i32)
    th2 = jnp.full((16,), 2 * SEG, _i32)
    th3 = jnp.full((16,), 3 * SEG, _i32)
    ones16 = jnp.full((16,), 1, _i32)
    zeros16 = jnp.full((16,), 0, _i32)

    pltpu.sync_copy(hist_ref, histbuf)

    def _z(i, carry):
        for j in range(8):
            zrow[i, pl.ds(j * 16, 16)] = jnp.zeros((16,), _f32)
        return carry
    lax.fori_loop(0, 128, _z, 0)

    def _zc(i, carry):
        zcnt[pl.ds(i * 16, 16)] = jnp.zeros((16,), _f32)
        return carry
    lax.fori_loop(0, 32, _zc, 0)

    def run_scoped_body(cst, cln):
        def own_store(cidx, tt, valid, run2, cap):
            @pl.when(tt == 0)
            def _():
                cst[cidx] = run2

            @pl.when(tt == 31)
            def _():
                cln[cidx] = run2 + cap - cst[cidx]
        _offsets(histbuf, own_store)

        def chunk_loop(q, carry):
            c = c2 * (NCH // 2) + q
            lov = jnp.full((16,), c * CH, _i32)
            hiv = jnp.full((16,), (c + 1) * CH, _i32)

            zd = []
            for k4 in range(4):
                zd.append(pltpu.async_copy(
                    zrow, sh_s.at[pl.ds(s * 512 + k4 * 128, 128)], zsem))
            zd.append(pltpu.async_copy(zcnt, sh_c.at[pl.ds(s * 512, 512)],
                                       zsem))
            for dsc in zd:
                dsc.wait()
            plsc.subcore_barrier()

            nw = cln[c] >> 6
            wlo = (nw * s) >> 4
            whi = (nw * (s + 1)) >> 4
            base = cst[c] + (wlo << 6)
            nwt = whi - wlo

            def wloop(w, carry2):
                off = pl.multiple_of(base + (w << 6), 64)
                pltpu.sync_copy(bkt_ref.at[pl.ds(off, 64)], pbuf)
                pltpu.async_copy(dst_ref.at[pbuf], dbuf, gsem).wait()
                for v in range(4):
                    sl = pl.ds(v * 16, 16)
                    pn = pbuf[sl]
                    d = dbuf[sl]
                    pv = (jnp.where(pn >= th1, ones16, zeros16)
                          + jnp.where(pn >= th2, ones16, zeros16)
                          + jnp.where(pn >= th3, ones16, zeros16))
                    comb = d + pv * N
                    m = (comb >= lov) & (comb < hiv)
                    dstlbuf[0, sl] = jnp.where(m, comb - lov,
                                               jnp.full((16,), CH, _i32))
                    rowibuf[sl] = pn - pv * 96
                    onesbuf[sl] = jnp.where(m, onesf, zerosf)
                pltpu.async_copy(text_ref.at[rowibuf], rdata, rsem).wait()
                d1 = pltpu.async_copy(rdata, sh_s.at[dstlbuf.at[0]], ssem,
                                      add=True)
                d2 = pltpu.async_copy(onesbuf, sh_c.at[dstlbuf.at[0]], osem,
                                      add=True)
                d1.wait()
                d2.wait()
                return carry2
            lax.fori_loop(0, nwt, wloop, 0)
            plsc.subcore_barrier()

            pltpu.sync_copy(sh_s.at[pl.ds(s * 512, 512)],
                            pool_ref.at[pl.ds(c * CH + s * 512, 512)])
            pltpu.sync_copy(sh_c.at[pl.ds(s * 512, 512)],
                            cnt_ref.at[pl.ds(c * CH + s * 512, 512)])
            plsc.subcore_barrier()
            return carry
        lax.fori_loop(0, NCH // 2, chunk_loop, 0)

    pl.run_scoped(run_scoped_body,
                  pltpu.SMEM((64,), _i32), pltpu.SMEM((64,), _i32))


def _gather_body(pool_ref, t5_ref, lt_ref, bv_ref, b2b_ref, tfb_ref,
                 out_ref,
                 ltb, bvb, bib, tib, i2b, i3b, i4b,
                 rb1, rb2, rb3, rb4, rb5, ob,
                 gsem, csem, wsem):
    c = lax.axis_index("c")
    s = lax.axis_index("s")
    w = c * 16 + s
    base_e = w * EPT

    pltpu.sync_copy(lt_ref.at[pl.ds(base_e, EPT)], ltb)
    pltpu.sync_copy(bv_ref.at[pl.ds(base_e, EPT)], bvb)

    for k in range(EPT // 128):
        pltpu.async_copy(b2b_ref.at[bvb.at[pl.ds(k * 128, 128)]],
                         bib.at[pl.ds(k * 128, 128)], csem)
    for k in range(EPT // 128):
        pltpu.make_async_copy(b2b_ref.at[bvb.at[pl.ds(0, 128)]],
                              bib.at[pl.ds(0, 128)], csem).wait()
    for k in range(EPT // 128):
        pltpu.async_copy(tfb_ref.at[bib.at[pl.ds(k * 128, 128)]],
                         tib.at[pl.ds(k * 128, 128)], csem)
    for k in range(EPT // 128):
        pltpu.make_async_copy(tfb_ref.at[bib.at[pl.ds(0, 128)]],
                              tib.at[pl.ds(0, 128)], csem).wait()

    def addoff(i, carry):
        sl = pl.ds(i * 16, 16)
        i2b[sl] = bvb[sl] + N
        i3b[sl] = bib[sl] + 2 * N
        i4b[sl] = ltb[sl] + 3 * N
        return carry
    lax.fori_loop(0, EPT // 16, addoff, 0)

    NB = EPT // GB

    def issue(b, par):
        sl = pl.ds(b * GB, GB)
        pltpu.async_copy(pool_ref.at[ltb.at[sl]], rb1.at[par], gsem)
        pltpu.async_copy(pool_ref.at[i2b.at[sl]], rb2.at[par], gsem)
        pltpu.async_copy(pool_ref.at[i3b.at[sl]], rb3.at[par], gsem)
        pltpu.async_copy(pool_ref.at[i4b.at[sl]], rb4.at[par], gsem)
        pltpu.async_copy(t5_ref.at[tib.at[sl]], rb5.at[par], gsem)

    issue(0, 0)

    def main(b, carry):
        par = lax.rem(b, 2)
        sl0 = pl.ds(0, GB)
        for rb in (rb1, rb2, rb3, rb4):
            pltpu.make_async_copy(pool_ref.at[ltb.at[sl0]], rb.at[par],
                                  gsem).wait()
        pltpu.make_async_copy(t5_ref.at[tib.at[sl0]], rb5.at[par], gsem).wait()

        @pl.when(b + 1 < NB)
        def _next():
            issue(b + 1, lax.rem(b + 1, 2))

        @pl.when(b >= 2)
        def _reuse():
            pltpu.make_async_copy(ob.at[par], out_ref.at[pl.ds(base_e, GB)],
                                  wsem).wait()

        def rowloop(r, carry2):
            for jj in range(8):
                slj = pl.ds(jj * 16, 16)
                ob[par, r, slj] = (rb1[par, r, slj] + rb2[par, r, slj]
                                   + rb3[par, r, slj] + rb4[par, r, slj]
                                   + rb5[par, r, slj])
            return carry2
        lax.fori_loop(0, GB, rowloop, 0)

        pltpu.async_copy(ob.at[par], out_ref.at[pl.ds(base_e + b * GB, GB)],
                         wsem)
        return carry
    lax.fori_loop(0, NB, main, 0)
    pltpu.make_async_copy(ob.at[0], out_ref.at[pl.ds(base_e, GB)], wsem).wait()
    pltpu.make_async_copy(ob.at[1], out_ref.at[pl.ds(base_e, GB)], wsem).wait()


def _sc_calls(text, dstx, t5, lt_p, bv_p, bv2b, tfb):
    mesh = plsc.VectorSubcoreMesh(core_axis_name="c", subcore_axis_name="s")
    hist = pl.kernel(
        _hist_body,
        out_type=jax.ShapeDtypeStruct((32 * NK * 16,), _i32),
        mesh=mesh,
        scratch_types=[
            pltpu.VMEM((TS,), _i32),
            pltpu.VMEM((16 * NK,), _i32),
        ],
    )(dstx)
    bkt = pl.kernel(
        _place_body,
        out_type=jax.ShapeDtypeStruct((BKTX,), _i32),
        mesh=mesh,
        scratch_types=[
            pltpu.VMEM((TS,), _i32),
            pltpu.VMEM((32 * NK * 16,), _i32),
            pltpu.VMEM((1, 128), _i32),
            pltpu.VMEM((128,), _i32),
            pltpu.SemaphoreType.DMA,
        ],
    )(dstx, hist)
    pool_sums, cnts = pl.kernel(
        _accum_body,
        out_type=[jax.ShapeDtypeStruct((POOL, D), _f32),
                  jax.ShapeDtypeStruct((POOL,), _f32)],
        mesh=mesh,
        scratch_types=[
            pltpu.VMEM_SHARED((CH + 16, D), _f32),
            pltpu.VMEM_SHARED((CH + 256,), _f32),
            pltpu.VMEM((32 * NK * 16,), _i32),
            pltpu.VMEM((64,), _i32),
            pltpu.VMEM((64,), _i32),
            pltpu.VMEM((64,), _i32),
            pltpu.VMEM((1, 64), _i32),
            pltpu.VMEM((64,), _f32),
            pltpu.VMEM((64, D), _f32),
            pltpu.VMEM((128, D), _f32),
            pltpu.VMEM((512,), _f32),
            pltpu.SemaphoreType.DMA,
            pltpu.SemaphoreType.DMA,
            pltpu.SemaphoreType.DMA,
            pltpu.SemaphoreType.DMA,
            pltpu.SemaphoreType.DMA,
        ],
    )(dstx, hist, bkt, text)
    pool = _divide(pool_sums, cnts)
    outp = pl.kernel(
        _gather_body,
        out_type=jax.ShapeDtypeStruct((EP, D), _f32),
        mesh=mesh,
        scratch_types=[
            pltpu.VMEM((EPT,), _i32),
            pltpu.VMEM((EPT,), _i32),
            pltpu.VMEM((EPT,), _i32),
            pltpu.VMEM((EPT,), _i32),
            pltpu.VMEM((EPT,), _i32),
            pltpu.VMEM((EPT,), _i32),
            pltpu.VMEM((EPT,), _i32),
            pltpu.VMEM((2, GB, D), _f32),
            pltpu.VMEM((2, GB, D), _f32),
            pltpu.VMEM((2, GB, D), _f32),
            pltpu.VMEM((2, GB, D), _f32),
            pltpu.VMEM((2, GB, D), _f32),
            pltpu.VMEM((2, GB, D), _f32),
            pltpu.SemaphoreType.DMA,
            pltpu.SemaphoreType.DMA,
            pltpu.SemaphoreType.DMA,
        ],
    )(pool, t5, lt_p, bv_p, bv2b, tfb)
    return outp


def kernel(h_legislator_term, h_bill_version, h_committee, h_topic, vote_edges,
           bv2b, topic_for_bill, prior_edge_src, read_edge_dst, member_edge_dst,
           W, b):
    lt_idx = vote_edges[0]
    bv_idx = vote_edges[1]

    segpad = jnp.full((SEG - N,), PAD_DST, _i32)
    dstx = jnp.concatenate([
        lt_idx, segpad, prior_edge_src, segpad,
        read_edge_dst, segpad, member_edge_dst, segpad,
        jnp.full((DSTX - E4P,), PAD_DST, _i32)])
    lt_p = jnp.pad(lt_idx, (0, EP - E))
    bv_p = jnp.pad(bv_idx, (0, EP - E))

    t4, t5 = _transforms(h_legislator_term, h_bill_version, h_committee,
                         h_topic, W, b)
    text = jnp.concatenate([t4.reshape(4 * N, D),
                            jnp.zeros((TEXT - 4 * N, D), _f32)])

    outp = _sc_calls(text, dstx, t5, lt_p, bv_p, bv2b, topic_for_bill)
    return outp[:E]


# speedup vs baseline: 3.9613x; 3.9613x over previous
import jax
import jax.numpy as jnp
from jax import lax
from jax.experimental import pallas as pl
from jax.experimental.pallas import tpu as pltpu
from jax.experimental.pallas import tpu_sc as plsc

N = 100000
E = 100000
D = 128
N_TOPIC = 10000

SEG = 100096
E4P = 4 * SEG
TS = E4P // 32
NV = TS // 16
CH = 8192
NCH = 50
POOL = NCH * CH
NK = 13
BKT = 32 * 13376
BKTX = BKT + 16
LB = 13376
LBX = LB + 16
PADPOS = E4P
DSTX = E4P + 64
TEXT = 400160
PAD_DST = 2_000_000
EP = 102400
EPT = EP // 32
GB = 64

_f32 = jnp.float32
_i32 = jnp.int32


def _transform_kernel(hl_ref, hb_ref, hc_ref, w_ref, out_ref):
    w = w_ref[...]
    dn = (((1,), (1,)), ((), ()))
    out_ref[0] = lax.dot_general(hl_ref[...], w[:, 0:128], dn)
    out_ref[1] = lax.dot_general(hb_ref[...], w[:, 128:256], dn)
    out_ref[2] = lax.dot_general(hc_ref[...], w[:, 256:384], dn)
    out_ref[3] = lax.dot_general(hc_ref[...], w[:, 384:512], dn)


def _topic_kernel(ht_ref, w_ref, b_ref, out_ref):
    dn = (((1,), (1,)), ((), ()))
    out_ref[...] = lax.dot_general(ht_ref[...], w_ref[...][:, 512:640], dn) + b_ref[...]


def _div_kernel(s_ref, c_ref, out_ref):
    out_ref[...] = s_ref[...] / jnp.maximum(c_ref[...], 1.0)[:, None]


def _transforms(h_lt, h_bv, h_c, h_topic, W, b):
    RB = 1000
    t4 = pl.pallas_call(
        _transform_kernel,
        grid=(N // RB,),
        in_specs=[
            pl.BlockSpec((RB, D), lambda i: (i, 0)),
            pl.BlockSpec((RB, D), lambda i: (i, 0)),
            pl.BlockSpec((RB, D), lambda i: (i, 0)),
            pl.BlockSpec((D, 5 * D), lambda i: (0, 0)),
        ],
        out_specs=pl.BlockSpec((4, RB, D), lambda i: (0, i, 0)),
        out_shape=jax.ShapeDtypeStruct((4, N, D), _f32),
    )(h_lt, h_bv, h_c, W)
    t5 = pl.pallas_call(
        _topic_kernel,
        grid=(N_TOPIC // RB,),
        in_specs=[
            pl.BlockSpec((RB, D), lambda i: (i, 0)),
            pl.BlockSpec((D, 5 * D), lambda i: (0, 0)),
            pl.BlockSpec((D,), lambda i: (0,)),
        ],
        out_specs=pl.BlockSpec((RB, D), lambda i: (i, 0)),
        out_shape=jax.ShapeDtypeStruct((N_TOPIC, D), _f32),
    )(h_topic, W, b)
    return t4, t5


def _divide(pool_sums, cnts):
    RB = 1024
    return pl.pallas_call(
        _div_kernel,
        grid=(POOL // RB,),
        in_specs=[
            pl.BlockSpec((RB, D), lambda i: (i, 0)),
            pl.BlockSpec((RB,), lambda i: (i,)),
        ],
        out_specs=pl.BlockSpec((RB, D), lambda i: (i, 0)),
        out_shape=jax.ShapeDtypeStruct((POOL, D), _f32),
    )(pool_sums, cnts)


def _hist_body(dst_ref, hist_ref, dstbuf, hrow):
    c2 = lax.axis_index("c")
    s = lax.axis_index("s")
    t = c2 * 16 + s
    p = t // 8
    poff = p * N
    cb = 12 * p
    ones16 = jnp.full((16,), 1, _i32)
    zeros16 = jnp.full((16,), 0, _i32)

    pltpu.sync_copy(dst_ref.at[pl.ds(t * TS, TS)], dstbuf)

    cbkv = [jnp.full((16,), cb + k, _i32) for k in range(NK)]

    def sb(i, cnts):
        d = dstbuf[pl.ds(i * 16, 16)]
        cid = lax.shift_right_logical(d + poff, 13)
        out = []
        for k in range(NK):
            eq = cid == cbkv[k]
            out.append(cnts[k] + jnp.where(eq, ones16, zeros16))
        return tuple(out)
    cnts = lax.fori_loop(0, NV, sb, tuple(zeros16 for _ in range(NK)))
    for k in range(NK):
        hrow[pl.ds(k * 16, 16)] = cnts[k]
    pltpu.sync_copy(hrow, hist_ref.at[pl.ds(t * 16 * NK, 16 * NK)])


def _offsets(histbuf, own_store):
    def cloop(cidx, run):
        def tloop(tt, run2):
            pp = tt // 8
            kk = cidx - 12 * pp
            valid = (kk >= 0) & (kk < NK)
            kk_c = jnp.clip(kk, 0, NK - 1)
            hv = histbuf[pl.ds((tt * NK + kk_c) * 16, 16)]
            sm = hv[0]
            for L in range(1, 16):
                sm = sm + hv[L]
            cap = jnp.where(valid, ((sm + 63) >> 6) << 6, 0)
            own_store(cidx, tt, valid, run2, cap)
            return run2 + cap
        return lax.fori_loop(0, 32, tloop, run)
    lax.fori_loop(0, NCH, cloop, jnp.array(0, _i32))


def _place_body(dst_ref, hist_ref, bkt_ref,
                dstbuf, histbuf, localbuf, bsem):
    c2 = lax.axis_index("c")
    s = lax.axis_index("s")
    t = c2 * 16 + s
    p = t // 8
    poff = p * N
    cb = 12 * p
    iota16 = lax.iota(_i32, 16)
    ones16 = jnp.full((16,), 1, _i32)
    zeros16 = jnp.full((16,), 0, _i32)

    pltpu.sync_copy(dst_ref.at[pl.ds(t * TS, TS)], dstbuf)
    pltpu.sync_copy(hist_ref, histbuf)

    padv = jnp.full((16,), PADPOS, _i32)

    def _pf(i, carry):
        localbuf[pl.ds(i * 16, 16)] = padv
        return carry
    lax.fori_loop(0, LBX // 16, _pf, 0)

    def run_scoped_body(mb):
        def own_store(cidx, tt, valid, run2, cap):
            @pl.when(valid & (tt == t))
            def _():
                mb[cidx - cb] = run2
        _offsets(histbuf, own_store)

        cbkv = [jnp.full((16,), cb + k, _i32) for k in range(NK)]
        trashp = jnp.full((16,), LB, _i32) + iota16
        base_pos = t * TS

        basev = []
        caps = []
        lbs = []
        lb = jnp.array(0, _i32)
        for k in range(NK):
            hv = histbuf[pl.ds((t * NK + k) * 16, 16)]
            bvec = jnp.full((16,), lb, _i32)
            run = lb
            for L in range(1, 16):
                run = run + hv[L - 1]
                bvec = jnp.where(iota16 == jnp.full((16,), L, _i32),
                                 jnp.full((16,), 0, _i32) + run, bvec)
            sm = run + hv[15] - lb
            cap = ((sm + 63) >> 6) << 6
            basev.append(bvec)
            caps.append(cap)
            lbs.append(lb)
            lb = lb + cap

        def do_vreg(i, ranks):
            d = dstbuf[pl.ds(i * 16, 16)]
            cid = lax.shift_right_logical(d + poff, 13)
            pos = trashp
            out = []
            for k in range(NK):
                eq = cid == cbkv[k]
                pos = jnp.where(eq, basev[k] + ranks[k], pos)
                out.append(ranks[k] + jnp.where(eq, ones16, zeros16))
            posn = base_pos + i * 16 + iota16
            plsc.store_scatter(localbuf, [pos], posn)
            return tuple(out)
        lax.fori_loop(0, NV, do_vreg, tuple(zeros16 for _ in range(NK)))

        nissued = jnp.array(0, _i32)
        for k in range(NK):
            gbase = mb[k]
            lbase = lbs[k]

            def wb(w, cc):
                so = pl.multiple_of(lbase + (w << 6), 64)
                do = pl.multiple_of(gbase + (w << 6), 64)
                pltpu.async_copy(localbuf.at[pl.ds(so, 64)],
                                 bkt_ref.at[pl.ds(do, 64)], bsem)
                return cc + 1
            nissued = lax.fori_loop(0, caps[k] >> 6, wb, nissued)

        def wtloop(i, carry):
            pltpu.make_async_copy(localbuf.at[pl.ds(0, 64)],
                                  bkt_ref.at[pl.ds(0, 64)], bsem).wait()
            return carry
        lax.fori_loop(0, nissued, wtloop, 0)

    pl.run_scoped(run_scoped_body, pltpu.SMEM((16,), _i32))


def _accum_body(dst_ref, hist_ref, bkt_ref, text_ref, pool_ref, cnt_ref,
                sh_s, sh_c,
                histbuf, pbuf, dbuf, rowibuf, dstlbuf, onesbuf,
                rdata, zrow, zcnt,
                gsem, rsem, ssem, osem, zsem):
    c2 = lax.axis_index("c")
    s = lax.axis_index("s")
    iota16 = lax.iota(_i32, 16)
    onesf = jnp.full((16,), 1.0, _f32)
    zerosf = jnp.full((16,), 0.0, _f32)
    th1 = jnp.full((16,), SEG, _i32)
    th2 = jnp.full((16,), 2 * SEG, _i32)
    th3 = jnp.full((16,), 3 * SEG, _i32)
    ones16 = jnp.full((16,), 1, _i32)
    zeros16 = jnp.full((16,), 0, _i32)

    pltpu.sync_copy(hist_ref, histbuf)

    def _z(i, carry):
        for j in range(8):
            zrow[i, pl.ds(j * 16, 16)] = jnp.zeros((16,), _f32)
        return carry
    lax.fori_loop(0, 128, _z, 0)

    def _zc(i, carry):
        zcnt[pl.ds(i * 16, 16)] = jnp.zeros((16,), _f32)
        return carry
    lax.fori_loop(0, 32, _zc, 0)

    def run_scoped_body(cst, cln):
        def own_store(cidx, tt, valid, run2, cap):
            @pl.when(tt == 0)
            def _():
                cst[cidx] = run2

            @pl.when(tt == 31)
            def _():
                cln[cidx] = run2 + cap - cst[cidx]
        _offsets(histbuf, own_store)

        def chunk_loop(q, carry):
            c = c2 * (NCH // 2) + q
            lov = jnp.full((16,), c * CH, _i32)
            hiv = jnp.full((16,), (c + 1) * CH, _i32)

            zd = []
            for k4 in range(4):
                zd.append(pltpu.async_copy(
                    zrow, sh_s.at[pl.ds(s * 512 + k4 * 128, 128)], zsem))
            zd.append(pltpu.async_copy(zcnt, sh_c.at[pl.ds(s * 512, 512)],
                                       zsem))
            for dsc in zd:
                dsc.wait()
            plsc.subcore_barrier()

            nw = cln[c] >> 6
            wlo = (nw * s) >> 4
            whi = (nw * (s + 1)) >> 4
            base = cst[c] + (wlo << 6)
            nwt = whi - wlo

            def wloop(w, carry2):
                off = pl.multiple_of(base + (w << 6), 64)
                pltpu.sync_copy(bkt_ref.at[pl.ds(off, 64)], pbuf)
                pltpu.async_copy(dst_ref.at[pbuf], dbuf, gsem).wait()
                for v in range(4):
                    sl = pl.ds(v * 16, 16)
                    pn = pbuf[sl]
                    d = dbuf[sl]
                    pv = (jnp.where(pn >= th1, ones16, zeros16)
                          + jnp.where(pn >= th2, ones16, zeros16)
                          + jnp.where(pn >= th3, ones16, zeros16))
                    comb = d + pv * N
                    m = (comb >= lov) & (comb < hiv)
                    dstlbuf[0, sl] = jnp.where(m, comb - lov,
                                               jnp.full((16,), CH, _i32))
                    rowibuf[sl] = pn - pv * 96
                    onesbuf[sl] = jnp.where(m, onesf, zerosf)
                pltpu.async_copy(text_ref.at[rowibuf], rdata, rsem).wait()
                d1 = pltpu.async_copy(rdata, sh_s.at[dstlbuf.at[0]], ssem,
                                      add=True)
                d2 = pltpu.async_copy(onesbuf, sh_c.at[dstlbuf.at[0]], osem,
                                      add=True)
                d1.wait()
                d2.wait()
                return carry2
            lax.fori_loop(0, nwt, wloop, 0)
            plsc.subcore_barrier()

            pltpu.sync_copy(sh_s.at[pl.ds(s * 512, 512)],
                            pool_ref.at[pl.ds(c * CH + s * 512, 512)])
            pltpu.sync_copy(sh_c.at[pl.ds(s * 512, 512)],
                            cnt_ref.at[pl.ds(c * CH + s * 512, 512)])
            plsc.subcore_barrier()
            return carry
        lax.fori_loop(0, NCH // 2, chunk_loop, 0)

    pl.run_scoped(run_scoped_body,
                  pltpu.SMEM((64,), _i32), pltpu.SMEM((64,), _i32))


def _gather_body(pool_ref, t5_ref, lt_ref, bv_ref, b2b_ref, tfb_ref,
                 out_ref,
                 ltb, bvb, bib, tib, i2b, i3b, i4b,
                 rb1, rb2, rb3, rb4, rb5, ob,
                 gsem, csem, wsem):
    c = lax.axis_index("c")
    s = lax.axis_index("s")
    w = c * 16 + s
    base_e = w * EPT

    pltpu.sync_copy(lt_ref.at[pl.ds(base_e, EPT)], ltb)
    pltpu.sync_copy(bv_ref.at[pl.ds(base_e, EPT)], bvb)

    for k in range(EPT // 128):
        pltpu.async_copy(b2b_ref.at[bvb.at[pl.ds(k * 128, 128)]],
                         bib.at[pl.ds(k * 128, 128)], csem)
    for k in range(EPT // 128):
        pltpu.make_async_copy(b2b_ref.at[bvb.at[pl.ds(0, 128)]],
                              bib.at[pl.ds(0, 128)], csem).wait()
    for k in range(EPT // 128):
        pltpu.async_copy(tfb_ref.at[bib.at[pl.ds(k * 128, 128)]],
                         tib.at[pl.ds(k * 128, 128)], csem)
    for k in range(EPT // 128):
        pltpu.make_async_copy(tfb_ref.at[bib.at[pl.ds(0, 128)]],
                              tib.at[pl.ds(0, 128)], csem).wait()

    def addoff(i, carry):
        sl = pl.ds(i * 16, 16)
        i2b[sl] = bvb[sl] + N
        i3b[sl] = bib[sl] + 2 * N
        i4b[sl] = ltb[sl] + 3 * N
        return carry
    lax.fori_loop(0, EPT // 16, addoff, 0)

    NB = EPT // GB

    def issue(b, par):
        sl = pl.ds(b * GB, GB)
        pltpu.async_copy(pool_ref.at[ltb.at[sl]], rb1.at[par], gsem)
        pltpu.async_copy(pool_ref.at[i2b.at[sl]], rb2.at[par], gsem)
        pltpu.async_copy(pool_ref.at[i3b.at[sl]], rb3.at[par], gsem)
        pltpu.async_copy(pool_ref.at[i4b.at[sl]], rb4.at[par], gsem)
        pltpu.async_copy(t5_ref.at[tib.at[sl]], rb5.at[par], gsem)

    issue(0, 0)

    def main(b, carry):
        par = lax.rem(b, 2)
        sl0 = pl.ds(0, GB)
        for rb in (rb1, rb2, rb3, rb4):
            pltpu.make_async_copy(pool_ref.at[ltb.at[sl0]], rb.at[par],
                                  gsem).wait()
        pltpu.make_async_copy(t5_ref.at[tib.at[sl0]], rb5.at[par], gsem).wait()

        @pl.when(b + 1 < NB)
        def _next():
            issue(b + 1, lax.rem(b + 1, 2))

        @pl.when(b >= 2)
        def _reuse():
            pltpu.make_async_copy(ob.at[par], out_ref.at[pl.ds(base_e, GB)],
                                  wsem).wait()

        def rowloop(r, carry2):
            for jj in range(8):
                slj = pl.ds(jj * 16, 16)
                ob[par, r, slj] = (rb1[par, r, slj] + rb2[par, r, slj]
                                   + rb3[par, r, slj] + rb4[par, r, slj]
                                   + rb5[par, r, slj])
            return carry2
        lax.fori_loop(0, GB, rowloop, 0)

        pltpu.async_copy(ob.at[par], out_ref.at[pl.ds(base_e + b * GB, GB)],
                         wsem)
        return carry
    lax.fori_loop(0, NB, main, 0)
    pltpu.make_async_copy(ob.at[0], out_ref.at[pl.ds(base_e, GB)], wsem).wait()
    pltpu.make_async_copy(ob.at[1], out_ref.at[pl.ds(base_e, GB)], wsem).wait()


def _sc_calls(text, dstx, t5, lt_p, bv_p, bv2b, tfb):
    mesh = plsc.VectorSubcoreMesh(core_axis_name="c", subcore_axis_name="s")
    hist = pl.kernel(
        _hist_body,
        out_type=jax.ShapeDtypeStruct((32 * NK * 16,), _i32),
        mesh=mesh,
        scratch_types=[
            pltpu.VMEM((TS,), _i32),
            pltpu.VMEM((16 * NK,), _i32),
        ],
    )(dstx)
    bkt = pl.kernel(
        _place_body,
        out_type=jax.ShapeDtypeStruct((BKTX,), _i32),
        mesh=mesh,
        compiler_params=pltpu.CompilerParams(needs_layout_passes=False),
        scratch_types=[
            pltpu.VMEM((TS,), _i32),
            pltpu.VMEM((32 * NK * 16,), _i32),
            pltpu.VMEM((LBX,), _i32),
            pltpu.SemaphoreType.DMA,
        ],
    )(dstx, hist)
    pool_sums, cnts = pl.kernel(
        _accum_body,
        out_type=[jax.ShapeDtypeStruct((POOL, D), _f32),
                  jax.ShapeDtypeStruct((POOL,), _f32)],
        mesh=mesh,
        scratch_types=[
            pltpu.VMEM_SHARED((CH + 16, D), _f32),
            pltpu.VMEM_SHARED((CH + 256,), _f32),
            pltpu.VMEM((32 * NK * 16,), _i32),
            pltpu.VMEM((64,), _i32),
            pltpu.VMEM((64,), _i32),
            pltpu.VMEM((64,), _i32),
            pltpu.VMEM((1, 64), _i32),
            pltpu.VMEM((64,), _f32),
            pltpu.VMEM((64, D), _f32),
            pltpu.VMEM((128, D), _f32),
            pltpu.VMEM((512,), _f32),
            pltpu.SemaphoreType.DMA,
            pltpu.SemaphoreType.DMA,
            pltpu.SemaphoreType.DMA,
            pltpu.SemaphoreType.DMA,
            pltpu.SemaphoreType.DMA,
        ],
    )(dstx, hist, bkt, text)
    pool = _divide(pool_sums, cnts)
    outp = pl.kernel(
        _gather_body,
        out_type=jax.ShapeDtypeStruct((EP, D), _f32),
        mesh=mesh,
        scratch_types=[
            pltpu.VMEM((EPT,), _i32),
            pltpu.VMEM((EPT,), _i32),
            pltpu.VMEM((EPT,), _i32),
            pltpu.VMEM((EPT,), _i32),
            pltpu.VMEM((EPT,), _i32),
            pltpu.VMEM((EPT,), _i32),
            pltpu.VMEM((EPT,), _i32),
            pltpu.VMEM((2, GB, D), _f32),
            pltpu.VMEM((2, GB, D), _f32),
            pltpu.VMEM((2, GB, D), _f32),
            pltpu.VMEM((2, GB, D), _f32),
            pltpu.VMEM((2, GB, D), _f32),
            pltpu.VMEM((2, GB, D), _f32),
            pltpu.SemaphoreType.DMA,
            pltpu.SemaphoreType.DMA,
            pltpu.SemaphoreType.DMA,
        ],
    )(pool, t5, lt_p, bv_p, bv2b, tfb)
    return outp


def kernel(h_legislator_term, h_bill_version, h_committee, h_topic, vote_edges,
           bv2b, topic_for_bill, prior_edge_src, read_edge_dst, member_edge_dst,
           W, b):
    lt_idx = vote_edges[0]
    bv_idx = vote_edges[1]

    segpad = jnp.full((SEG - N,), PAD_DST, _i32)
    dstx = jnp.concatenate([
        lt_idx, segpad, prior_edge_src, segpad,
        read_edge_dst, segpad, member_edge_dst, segpad,
        jnp.full((DSTX - E4P,), PAD_DST, _i32)])
    lt_p = jnp.pad(lt_idx, (0, EP - E))
    bv_p = jnp.pad(bv_idx, (0, EP - E))

    t4, t5 = _transforms(h_legislator_term, h_bill_version, h_committee,
                         h_topic, W, b)
    text = jnp.concatenate([t4.reshape(4 * N, D),
                            jnp.zeros((TEXT - 4 * N, D), _f32)])

    outp = _sc_calls(text, dstx, t5, lt_p, bv_p, bv2b, topic_for_bill)
    return outp[:E]


# speedup vs baseline: 4.1498x; 1.0476x over previous
import jax
import jax.numpy as jnp
from jax import lax
from jax.experimental import pallas as pl
from jax.experimental.pallas import tpu as pltpu
from jax.experimental.pallas import tpu_sc as plsc

N = 100000
E = 100000
D = 128
N_TOPIC = 10000

SEG = 100096
E4P = 4 * SEG
TS = E4P // 32
NV = TS // 16
CH = 8192
NCH = 50
POOL = NCH * CH
NK = 13
BKT = 32 * 13376
BKTX = BKT + 16
LB = 13376
LBX = LB + 16
PADPOS = E4P
DSTX = E4P + 64
TEXT = 400160
PAD_DST = 2_000_000
EP = 102400
EPT = EP // 32
GB = 64

_f32 = jnp.float32
_i32 = jnp.int32


def _transform_kernel(hl_ref, hb_ref, hc_ref, w_ref, out_ref):
    w = w_ref[...]
    dn = (((1,), (1,)), ((), ()))
    out_ref[0] = lax.dot_general(hl_ref[...], w[:, 0:128], dn)
    out_ref[1] = lax.dot_general(hb_ref[...], w[:, 128:256], dn)
    out_ref[2] = lax.dot_general(hc_ref[...], w[:, 256:384], dn)
    out_ref[3] = lax.dot_general(hc_ref[...], w[:, 384:512], dn)


def _topic_kernel(ht_ref, w_ref, b_ref, out_ref):
    dn = (((1,), (1,)), ((), ()))
    out_ref[...] = lax.dot_general(ht_ref[...], w_ref[...][:, 512:640], dn) + b_ref[...]


def _div_kernel(s_ref, c_ref, out_ref):
    out_ref[...] = s_ref[...] / jnp.maximum(c_ref[...], 1.0)[:, None]


def _transforms(h_lt, h_bv, h_c, h_topic, W, b):
    RB = 1000
    t4 = pl.pallas_call(
        _transform_kernel,
        grid=(N // RB,),
        in_specs=[
            pl.BlockSpec((RB, D), lambda i: (i, 0)),
            pl.BlockSpec((RB, D), lambda i: (i, 0)),
            pl.BlockSpec((RB, D), lambda i: (i, 0)),
            pl.BlockSpec((D, 5 * D), lambda i: (0, 0)),
        ],
        out_specs=pl.BlockSpec((4, RB, D), lambda i: (0, i, 0)),
        out_shape=jax.ShapeDtypeStruct((4, N, D), _f32),
    )(h_lt, h_bv, h_c, W)
    t5 = pl.pallas_call(
        _topic_kernel,
        grid=(N_TOPIC // RB,),
        in_specs=[
            pl.BlockSpec((RB, D), lambda i: (i, 0)),
            pl.BlockSpec((D, 5 * D), lambda i: (0, 0)),
            pl.BlockSpec((D,), lambda i: (0,)),
        ],
        out_specs=pl.BlockSpec((RB, D), lambda i: (i, 0)),
        out_shape=jax.ShapeDtypeStruct((N_TOPIC, D), _f32),
    )(h_topic, W, b)
    return t4, t5


def _divide(pool_sums, cnts):
    RB = 1024
    return pl.pallas_call(
        _div_kernel,
        grid=(POOL // RB,),
        in_specs=[
            pl.BlockSpec((RB, D), lambda i: (i, 0)),
            pl.BlockSpec((RB,), lambda i: (i,)),
        ],
        out_specs=pl.BlockSpec((RB, D), lambda i: (i, 0)),
        out_shape=jax.ShapeDtypeStruct((POOL, D), _f32),
    )(pool_sums, cnts)


def _hist_body(dst_ref, hist_ref, dstbuf, hrow):
    c2 = lax.axis_index("c")
    s = lax.axis_index("s")
    t = c2 * 16 + s
    p = t // 8
    poff = p * N
    cb = 12 * p
    ones16 = jnp.full((16,), 1, _i32)
    zeros16 = jnp.full((16,), 0, _i32)

    pltpu.sync_copy(dst_ref.at[pl.ds(t * TS, TS)], dstbuf)

    cbkv = [jnp.full((16,), cb + k, _i32) for k in range(NK)]

    def sb(i, cnts):
        d = dstbuf[pl.ds(i * 16, 16)]
        cid = lax.shift_right_logical(d + poff, 13)
        out = []
        for k in range(NK):
            eq = cid == cbkv[k]
            out.append(cnts[k] + jnp.where(eq, ones16, zeros16))
        return tuple(out)
    cnts = lax.fori_loop(0, NV, sb, tuple(zeros16 for _ in range(NK)))
    for k in range(NK):
        hrow[pl.ds(k * 16, 16)] = cnts[k]
    pltpu.sync_copy(hrow, hist_ref.at[pl.ds(t * 16 * NK, 16 * NK)])


def _offsets(histbuf, own_store):
    def cloop(cidx, run):
        def tloop(tt, run2):
            pp = tt // 8
            kk = cidx - 12 * pp
            valid = (kk >= 0) & (kk < NK)
            kk_c = jnp.clip(kk, 0, NK - 1)
            hv = histbuf[pl.ds((tt * NK + kk_c) * 16, 16)]
            sm = hv[0]
            for L in range(1, 16):
                sm = sm + hv[L]
            cap = jnp.where(valid, ((sm + 63) >> 6) << 6, 0)
            own_store(cidx, tt, valid, run2, cap)
            return run2 + cap
        return lax.fori_loop(0, 32, tloop, run)
    lax.fori_loop(0, NCH, cloop, jnp.array(0, _i32))


def _place_body(dst_ref, hist_ref, bkt_ref,
                dstbuf, histbuf, localbuf, bsem):
    c2 = lax.axis_index("c")
    s = lax.axis_index("s")
    t = c2 * 16 + s
    p = t // 8
    poff = p * N
    cb = 12 * p
    iota16 = lax.iota(_i32, 16)
    ones16 = jnp.full((16,), 1, _i32)
    zeros16 = jnp.full((16,), 0, _i32)

    pltpu.sync_copy(dst_ref.at[pl.ds(t * TS, TS)], dstbuf)
    pltpu.sync_copy(hist_ref, histbuf)

    padv = jnp.full((16,), PADPOS, _i32)

    def _pf(i, carry):
        localbuf[pl.ds(i * 16, 16)] = padv
        return carry
    lax.fori_loop(0, LBX // 16, _pf, 0)

    def run_scoped_body(mb):
        def own_store(cidx, tt, valid, run2, cap):
            @pl.when(valid & (tt == t))
            def _():
                mb[cidx - cb] = run2
        _offsets(histbuf, own_store)

        cbkv = [jnp.full((16,), cb + k, _i32) for k in range(NK)]
        trashp = jnp.full((16,), LB, _i32) + iota16
        base_pos = t * TS

        basev = []
        caps = []
        lbs = []
        lb = jnp.array(0, _i32)
        for k in range(NK):
            hv = histbuf[pl.ds((t * NK + k) * 16, 16)]
            bvec = jnp.full((16,), lb, _i32)
            run = lb
            for L in range(1, 16):
                run = run + hv[L - 1]
                bvec = jnp.where(iota16 == jnp.full((16,), L, _i32),
                                 jnp.full((16,), 0, _i32) + run, bvec)
            sm = run + hv[15] - lb
            cap = ((sm + 63) >> 6) << 6
            basev.append(bvec)
            caps.append(cap)
            lbs.append(lb)
            lb = lb + cap

        def do_vreg(i, ranks):
            d = dstbuf[pl.ds(i * 16, 16)]
            cid = lax.shift_right_logical(d + poff, 13)
            pos = trashp
            out = []
            for k in range(NK):
                eq = cid == cbkv[k]
                pos = jnp.where(eq, basev[k] + ranks[k], pos)
                out.append(ranks[k] + jnp.where(eq, ones16, zeros16))
            posn = base_pos + i * 16 + iota16
            plsc.store_scatter(localbuf, [pos], posn)
            return tuple(out)
        lax.fori_loop(0, NV, do_vreg, tuple(zeros16 for _ in range(NK)))

        nissued = jnp.array(0, _i32)
        for k in range(NK):
            gbase = mb[k]
            lbase = lbs[k]

            def wb(w, cc):
                so = pl.multiple_of(lbase + (w << 6), 64)
                do = pl.multiple_of(gbase + (w << 6), 64)
                pltpu.async_copy(localbuf.at[pl.ds(so, 64)],
                                 bkt_ref.at[pl.ds(do, 64)], bsem)
                return cc + 1
            nissued = lax.fori_loop(0, caps[k] >> 6, wb, nissued)

        def wtloop(i, carry):
            pltpu.make_async_copy(localbuf.at[pl.ds(0, 64)],
                                  bkt_ref.at[pl.ds(0, 64)], bsem).wait()
            return carry
        lax.fori_loop(0, nissued, wtloop, 0)

    pl.run_scoped(run_scoped_body, pltpu.SMEM((16,), _i32))


def _accum_body(dst_ref, hist_ref, bkt_ref, text_ref, pool_ref, cnt_ref,
                sh_s, sh_c,
                histbuf, pbuf, dbuf, rowibufA, rowibufB,
                dstlbufA, dstlbufB, onesbufA, onesbufB,
                rdataA, rdataB, zrow, zcnt,
                gsem, rsemA, rsemB, ssemA, ssemB, osemA, osemB, zsem):
    c2 = lax.axis_index("c")
    s = lax.axis_index("s")
    iota16 = lax.iota(_i32, 16)
    onesf = jnp.full((16,), 1.0, _f32)
    zerosf = jnp.full((16,), 0.0, _f32)
    th1 = jnp.full((16,), SEG, _i32)
    th2 = jnp.full((16,), 2 * SEG, _i32)
    th3 = jnp.full((16,), 3 * SEG, _i32)
    ones16 = jnp.full((16,), 1, _i32)
    zeros16 = jnp.full((16,), 0, _i32)

    pltpu.sync_copy(hist_ref, histbuf)

    def _z(i, carry):
        for j in range(8):
            zrow[i, pl.ds(j * 16, 16)] = jnp.zeros((16,), _f32)
        return carry
    lax.fori_loop(0, 128, _z, 0)

    def _zc(i, carry):
        zcnt[pl.ds(i * 16, 16)] = jnp.zeros((16,), _f32)
        return carry
    lax.fori_loop(0, 32, _zc, 0)

    def run_scoped_body(cst, cln):
        def own_store(cidx, tt, valid, run2, cap):
            @pl.when(tt == 0)
            def _():
                cst[cidx] = run2

            @pl.when(tt == 31)
            def _():
                cln[cidx] = run2 + cap - cst[cidx]
        _offsets(histbuf, own_store)

        def chunk_loop(q, carry):
            c = c2 * (NCH // 2) + q
            lov = jnp.full((16,), c * CH, _i32)
            hiv = jnp.full((16,), (c + 1) * CH, _i32)

            zd = []
            for k4 in range(4):
                zd.append(pltpu.async_copy(
                    zrow, sh_s.at[pl.ds(s * 512 + k4 * 128, 128)], zsem))
            zd.append(pltpu.async_copy(zcnt, sh_c.at[pl.ds(s * 512, 512)],
                                       zsem))
            for dsc in zd:
                dsc.wait()
            plsc.subcore_barrier()

            nw = cln[c] >> 6
            wlo = (nw * s) >> 4
            whi = (nw * (s + 1)) >> 4
            base = cst[c] + (wlo << 6)
            nwt = whi - wlo

            def front(off, rowibuf, dstlbuf, onesbuf, rdata, rsem):
                pltpu.sync_copy(bkt_ref.at[pl.ds(off, 64)], pbuf)
                pltpu.async_copy(dst_ref.at[pbuf], dbuf, gsem).wait()
                for v in range(4):
                    sl = pl.ds(v * 16, 16)
                    pn = pbuf[sl]
                    d = dbuf[sl]
                    pv = (jnp.where(pn >= th1, ones16, zeros16)
                          + jnp.where(pn >= th2, ones16, zeros16)
                          + jnp.where(pn >= th3, ones16, zeros16))
                    comb = d + pv * N
                    m = (comb >= lov) & (comb < hiv)
                    dstlbuf[0, sl] = jnp.where(m, comb - lov,
                                               jnp.full((16,), CH, _i32))
                    rowibuf[sl] = pn - pv * 96
                    onesbuf[sl] = jnp.where(m, onesf, zerosf)
                pltpu.async_copy(text_ref.at[rowibuf], rdata, rsem)

            def back(rowibuf, dstlbuf, onesbuf, rdata, rsem, ssem, osem):
                pltpu.make_async_copy(text_ref.at[rowibuf], rdata, rsem).wait()
                pltpu.async_copy(rdata, sh_s.at[dstlbuf.at[0]], ssem, add=True)
                pltpu.async_copy(onesbuf, sh_c.at[dstlbuf.at[0]], osem,
                                 add=True)

            def wait_sc(dstlbuf, onesbuf, rdata, ssem, osem):
                pltpu.make_async_copy(rdata, sh_s.at[dstlbuf.at[0]],
                                      ssem).wait()
                pltpu.make_async_copy(onesbuf, sh_c.at[dstlbuf.at[0]],
                                      osem).wait()

            npair = nwt >> 1

            def ploop(h, carry2):
                @pl.when(h > 0)
                def _():
                    wait_sc(dstlbufA, onesbufA, rdataA, ssemA, osemA)
                    wait_sc(dstlbufB, onesbufB, rdataB, ssemB, osemB)
                offa = pl.multiple_of(base + ((h << 1) << 6), 64)
                offb = pl.multiple_of(base + (((h << 1) + 1) << 6), 64)
                front(offa, rowibufA, dstlbufA, onesbufA, rdataA, rsemA)
                front(offb, rowibufB, dstlbufB, onesbufB, rdataB, rsemB)
                back(rowibufA, dstlbufA, onesbufA, rdataA, rsemA, ssemA, osemA)
                back(rowibufB, dstlbufB, onesbufB, rdataB, rsemB, ssemB, osemB)
                return carry2
            lax.fori_loop(0, npair, ploop, 0)

            @pl.when(npair > 0)
            def _():
                wait_sc(dstlbufA, onesbufA, rdataA, ssemA, osemA)
                wait_sc(dstlbufB, onesbufB, rdataB, ssemB, osemB)

            @pl.when((nwt & 1) == 1)
            def _():
                offt = pl.multiple_of(base + ((nwt - 1) << 6), 64)
                front(offt, rowibufA, dstlbufA, onesbufA, rdataA, rsemA)
                back(rowibufA, dstlbufA, onesbufA, rdataA, rsemA, ssemA, osemA)
                wait_sc(dstlbufA, onesbufA, rdataA, ssemA, osemA)

            plsc.subcore_barrier()

            pltpu.sync_copy(sh_s.at[pl.ds(s * 512, 512)],
                            pool_ref.at[pl.ds(c * CH + s * 512, 512)])
            pltpu.sync_copy(sh_c.at[pl.ds(s * 512, 512)],
                            cnt_ref.at[pl.ds(c * CH + s * 512, 512)])
            plsc.subcore_barrier()
            return carry
        lax.fori_loop(0, NCH // 2, chunk_loop, 0)

    pl.run_scoped(run_scoped_body,
                  pltpu.SMEM((64,), _i32), pltpu.SMEM((64,), _i32))


def _gather_body(pool_ref, t5_ref, lt_ref, bv_ref, b2b_ref, tfb_ref,
                 out_ref,
                 ltb, bvb, bib, tib, i2b, i3b, i4b,
                 rb1, rb2, rb3, rb4, rb5, ob,
                 gsem, csem, wsem):
    c = lax.axis_index("c")
    s = lax.axis_index("s")
    w = c * 16 + s
    base_e = w * EPT

    pltpu.sync_copy(lt_ref.at[pl.ds(base_e, EPT)], ltb)
    pltpu.sync_copy(bv_ref.at[pl.ds(base_e, EPT)], bvb)

    for k in range(EPT // 128):
        pltpu.async_copy(b2b_ref.at[bvb.at[pl.ds(k * 128, 128)]],
                         bib.at[pl.ds(k * 128, 128)], csem)
    for k in range(EPT // 128):
        pltpu.make_async_copy(b2b_ref.at[bvb.at[pl.ds(0, 128)]],
                              bib.at[pl.ds(0, 128)], csem).wait()
    for k in range(EPT // 128):
        pltpu.async_copy(tfb_ref.at[bib.at[pl.ds(k * 128, 128)]],
                         tib.at[pl.ds(k * 128, 128)], csem)
    for k in range(EPT // 128):
        pltpu.make_async_copy(tfb_ref.at[bib.at[pl.ds(0, 128)]],
                              tib.at[pl.ds(0, 128)], csem).wait()

    def addoff(i, carry):
        sl = pl.ds(i * 16, 16)
        i2b[sl] = bvb[sl] + N
        i3b[sl] = bib[sl] + 2 * N
        i4b[sl] = ltb[sl] + 3 * N
        return carry
    lax.fori_loop(0, EPT // 16, addoff, 0)

    NB = EPT // GB

    def issue(b, par):
        sl = pl.ds(b * GB, GB)
        pltpu.async_copy(pool_ref.at[ltb.at[sl]], rb1.at[par], gsem)
        pltpu.async_copy(pool_ref.at[i2b.at[sl]], rb2.at[par], gsem)
        pltpu.async_copy(pool_ref.at[i3b.at[sl]], rb3.at[par], gsem)
        pltpu.async_copy(pool_ref.at[i4b.at[sl]], rb4.at[par], gsem)
        pltpu.async_copy(t5_ref.at[tib.at[sl]], rb5.at[par], gsem)

    issue(0, 0)

    def main(b, carry):
        par = lax.rem(b, 2)
        sl0 = pl.ds(0, GB)
        for rb in (rb1, rb2, rb3, rb4):
            pltpu.make_async_copy(pool_ref.at[ltb.at[sl0]], rb.at[par],
                                  gsem).wait()
        pltpu.make_async_copy(t5_ref.at[tib.at[sl0]], rb5.at[par], gsem).wait()

        @pl.when(b + 1 < NB)
        def _next():
            issue(b + 1, lax.rem(b + 1, 2))

        @pl.when(b >= 2)
        def _reuse():
            pltpu.make_async_copy(ob.at[par], out_ref.at[pl.ds(base_e, GB)],
                                  wsem).wait()

        def rowloop(r, carry2):
            for jj in range(8):
                slj = pl.ds(jj * 16, 16)
                ob[par, r, slj] = (rb1[par, r, slj] + rb2[par, r, slj]
                                   + rb3[par, r, slj] + rb4[par, r, slj]
                                   + rb5[par, r, slj])
            return carry2
        lax.fori_loop(0, GB, rowloop, 0)

        pltpu.async_copy(ob.at[par], out_ref.at[pl.ds(base_e + b * GB, GB)],
                         wsem)
        return carry
    lax.fori_loop(0, NB, main, 0)
    pltpu.make_async_copy(ob.at[0], out_ref.at[pl.ds(base_e, GB)], wsem).wait()
    pltpu.make_async_copy(ob.at[1], out_ref.at[pl.ds(base_e, GB)], wsem).wait()


def _sc_calls(text, dstx, t5, lt_p, bv_p, bv2b, tfb):
    mesh = plsc.VectorSubcoreMesh(core_axis_name="c", subcore_axis_name="s")
    hist = pl.kernel(
        _hist_body,
        out_type=jax.ShapeDtypeStruct((32 * NK * 16,), _i32),
        mesh=mesh,
        scratch_types=[
            pltpu.VMEM((TS,), _i32),
            pltpu.VMEM((16 * NK,), _i32),
        ],
    )(dstx)
    bkt = pl.kernel(
        _place_body,
        out_type=jax.ShapeDtypeStruct((BKTX,), _i32),
        mesh=mesh,
        compiler_params=pltpu.CompilerParams(needs_layout_passes=False),
        scratch_types=[
            pltpu.VMEM((TS,), _i32),
            pltpu.VMEM((32 * NK * 16,), _i32),
            pltpu.VMEM((LBX,), _i32),
            pltpu.SemaphoreType.DMA,
        ],
    )(dstx, hist)
    pool_sums, cnts = pl.kernel(
        _accum_body,
        out_type=[jax.ShapeDtypeStruct((POOL, D), _f32),
                  jax.ShapeDtypeStruct((POOL,), _f32)],
        mesh=mesh,
        scratch_types=[
            pltpu.VMEM_SHARED((CH + 16, D), _f32),
            pltpu.VMEM_SHARED((CH + 256,), _f32),
            pltpu.VMEM((32 * NK * 16,), _i32),
            pltpu.VMEM((64,), _i32),
            pltpu.VMEM((64,), _i32),
            pltpu.VMEM((64,), _i32),
            pltpu.VMEM((64,), _i32),
            pltpu.VMEM((1, 64), _i32),
            pltpu.VMEM((1, 64), _i32),
            pltpu.VMEM((64,), _f32),
            pltpu.VMEM((64,), _f32),
            pltpu.VMEM((64, D), _f32),
            pltpu.VMEM((64, D), _f32),
            pltpu.VMEM((128, D), _f32),
            pltpu.VMEM((512,), _f32),
            pltpu.SemaphoreType.DMA,
            pltpu.SemaphoreType.DMA,
            pltpu.SemaphoreType.DMA,
            pltpu.SemaphoreType.DMA,
            pltpu.SemaphoreType.DMA,
            pltpu.SemaphoreType.DMA,
            pltpu.SemaphoreType.DMA,
            pltpu.SemaphoreType.DMA,
        ],
    )(dstx, hist, bkt, text)
    pool = _divide(pool_sums, cnts)
    outp = pl.kernel(
        _gather_body,
        out_type=jax.ShapeDtypeStruct((EP, D), _f32),
        mesh=mesh,
        scratch_types=[
            pltpu.VMEM((EPT,), _i32),
            pltpu.VMEM((EPT,), _i32),
            pltpu.VMEM((EPT,), _i32),
            pltpu.VMEM((EPT,), _i32),
            pltpu.VMEM((EPT,), _i32),
            pltpu.VMEM((EPT,), _i32),
            pltpu.VMEM((EPT,), _i32),
            pltpu.VMEM((2, GB, D), _f32),
            pltpu.VMEM((2, GB, D), _f32),
            pltpu.VMEM((2, GB, D), _f32),
            pltpu.VMEM((2, GB, D), _f32),
            pltpu.VMEM((2, GB, D), _f32),
            pltpu.VMEM((2, GB, D), _f32),
            pltpu.SemaphoreType.DMA,
            pltpu.SemaphoreType.DMA,
            pltpu.SemaphoreType.DMA,
        ],
    )(pool, t5, lt_p, bv_p, bv2b, tfb)
    return outp


def kernel(h_legislator_term, h_bill_version, h_committee, h_topic, vote_edges,
           bv2b, topic_for_bill, prior_edge_src, read_edge_dst, member_edge_dst,
           W, b):
    lt_idx = vote_edges[0]
    bv_idx = vote_edges[1]

    segpad = jnp.full((SEG - N,), PAD_DST, _i32)
    dstx = jnp.concatenate([
        lt_idx, segpad, prior_edge_src, segpad,
        read_edge_dst, segpad, member_edge_dst, segpad,
        jnp.full((DSTX - E4P,), PAD_DST, _i32)])
    lt_p = jnp.pad(lt_idx, (0, EP - E))
    bv_p = jnp.pad(bv_idx, (0, EP - E))

    t4, t5 = _transforms(h_legislator_term, h_bill_version, h_committee,
                         h_topic, W, b)
    text = jnp.concatenate([t4.reshape(4 * N, D),
                            jnp.zeros((TEXT - 4 * N, D), _f32)])

    outp = _sc_calls(text, dstx, t5, lt_p, bv_p, bv2b, topic_for_bill)
    return outp[:E]
